# initial kernel scaffold (unmeasured)
import jax
import jax.numpy as jnp
from jax import lax
from jax.experimental import pallas as pl
from jax.experimental.pallas import tpu as pltpu

M, D = 8192, 2048
BR = 1024
R = M // BR


def kernel(partial, resid, gamma):
    p = partial.reshape(M, D).astype(jnp.bfloat16)
    gamma2 = gamma.reshape(1, D)

    def body(p_ref, resid_ref, gamma_ref, out_ref,
             recv_ref, send_sem, recv_sem, credit_sem):
        i = pl.program_id(0)
        my_x = lax.axis_index("x")
        my_y = lax.axis_index("y")
        my_z = lax.axis_index("z")
        partner = (1 - my_x, my_y, my_z)

        @pl.when(i == 0)
        def _():
            bar = pltpu.get_barrier_semaphore()
            pl.semaphore_signal(bar, inc=1, device_id=partner,
                                device_id_type=pl.DeviceIdType.MESH)
            pl.semaphore_wait(bar, 1)

        @pl.when(i > 0)
        def _():
            pl.semaphore_wait(credit_sem, 1)

        rdma = pltpu.make_async_remote_copy(
            src_ref=p_ref, dst_ref=recv_ref,
            send_sem=send_sem, recv_sem=recv_sem,
            device_id=partner, device_id_type=pl.DeviceIdType.MESH)
        rdma.start()
        local = p_ref[...].astype(jnp.float32) + resid_ref[...]
        rdma.wait()
        y = local + recv_ref[...].astype(jnp.float32)
        ms = jnp.mean(y * y, axis=-1, keepdims=True)
        out_ref[...] = y * lax.rsqrt(ms + 1e-6) * gamma_ref[...]

        @pl.when(i < R - 1)
        def _():
            pl.semaphore_signal(credit_sem, inc=1, device_id=partner,
                                device_id_type=pl.DeviceIdType.MESH)

    return pl.pallas_call(
        body,
        grid=(R,),
        out_shape=jax.ShapeDtypeStruct((M, D), jnp.float32),
        in_specs=[
            pl.BlockSpec((BR, D), lambda i: (i, 0)),
            pl.BlockSpec((BR, D), lambda i: (i, 0)),
            pl.BlockSpec((1, D), lambda i: (0, 0)),
        ],
        out_specs=pl.BlockSpec((BR, D), lambda i: (i, 0)),
        scratch_shapes=[
            pltpu.VMEM((BR, D), jnp.bfloat16),
            pltpu.SemaphoreType.DMA,
            pltpu.SemaphoreType.DMA,
            pltpu.SemaphoreType.REGULAR,
        ],
        compiler_params=pltpu.CompilerParams(collective_id=0),
    )(p, resid, gamma2)


# baseline (device time: 498279 ns/iter reference)
import jax
import jax.numpy as jnp
from jax import lax
from jax.experimental import pallas as pl
from jax.experimental.pallas import tpu as pltpu

M, D = 8192, 2048
BR = 512
R = M // BR


def kernel(partial, resid, gamma):
    p = partial.reshape(M, D).astype(jnp.bfloat16)
    gamma2 = gamma.reshape(1, D)

    def body(p_ref, resid_ref, gamma_ref, out_ref,
             recv_ref, send_sem, recv_sem, credit_sem):
        i = pl.program_id(0)
        my_x = lax.axis_index("x")
        my_y = lax.axis_index("y")
        my_z = lax.axis_index("z")
        partner = (1 - my_x, my_y, my_z)

        @pl.when(i == 0)
        def _():
            bar = pltpu.get_barrier_semaphore()
            pl.semaphore_signal(bar, inc=1, device_id=partner,
                                device_id_type=pl.DeviceIdType.MESH)
            pl.semaphore_wait(bar, 1)

        @pl.when(i > 0)
        def _():
            pl.semaphore_wait(credit_sem, 1)

        rdma = pltpu.make_async_remote_copy(
            src_ref=p_ref, dst_ref=recv_ref,
            send_sem=send_sem, recv_sem=recv_sem,
            device_id=partner, device_id_type=pl.DeviceIdType.MESH)
        rdma.start()
        local = p_ref[...].astype(jnp.float32) + resid_ref[...]
        rdma.wait()
        y = local + recv_ref[...].astype(jnp.float32)
        ms = jnp.mean(y * y, axis=-1, keepdims=True)
        out_ref[...] = y * lax.rsqrt(ms + 1e-6) * gamma_ref[...]

        @pl.when(i < R - 1)
        def _():
            pl.semaphore_signal(credit_sem, inc=1, device_id=partner,
                                device_id_type=pl.DeviceIdType.MESH)

    return pl.pallas_call(
        body,
        grid=(R,),
        out_shape=jax.ShapeDtypeStruct((M, D), jnp.float32),
        in_specs=[
            pl.BlockSpec((BR, D), lambda i: (i, 0)),
            pl.BlockSpec((BR, D), lambda i: (i, 0)),
            pl.BlockSpec((1, D), lambda i: (0, 0)),
        ],
        out_specs=pl.BlockSpec((BR, D), lambda i: (i, 0)),
        scratch_shapes=[
            pltpu.VMEM((BR, D), jnp.bfloat16),
            pltpu.SemaphoreType.DMA,
            pltpu.SemaphoreType.DMA,
            pltpu.SemaphoreType.REGULAR,
        ],
        compiler_params=pltpu.CompilerParams(collective_id=0),
    )(p, resid, gamma2)


# device time: 328398 ns/iter; 1.5173x vs baseline; 1.5173x over previous
import jax
import jax.numpy as jnp
from jax import lax
from jax.experimental import pallas as pl
from jax.experimental.pallas import tpu as pltpu

M, D = 8192, 2048
BR = 512
R = M // BR
H = BR // 2


def kernel(partial, resid, gamma):
    p = partial.reshape(M, D).astype(jnp.bfloat16)
    gamma2 = gamma.reshape(1, D)

    def body(ps_ref, pc_ref, resid_ref, gamma_ref, out_ref,
             xrecv, yrecv, x_send_sem, x_recv_sems, y_send_sem, y_recv_sems,
             x_credit, y_credit):
        i = pl.program_id(0)
        my_x = lax.axis_index("x")
        my_y = lax.axis_index("y")
        my_z = lax.axis_index("z")
        xpartner = (1 - my_x, my_y, my_z)
        ypartner = (my_x, 1 - my_y, my_z)

        s_i = i % 2
        s_p = (i - 1) % 2
        d_off = my_y * H
        f_off = (1 - my_y) * H

        @pl.when(i == 0)
        def _():
            bar = pltpu.get_barrier_semaphore()
            for nbr in (xpartner, ypartner):
                pl.semaphore_signal(bar, inc=1, device_id=nbr,
                                    device_id_type=pl.DeviceIdType.MESH)
            pl.semaphore_wait(bar, 2)

        @pl.when(jnp.logical_and(i < R, i >= 2))
        def _():
            pl.semaphore_wait(x_credit, 1)

        x_rdma = pltpu.make_async_remote_copy(
            src_ref=ps_ref.at[pl.ds(d_off, H), :],
            dst_ref=xrecv.at[s_i],
            send_sem=x_send_sem, recv_sem=x_recv_sems.at[s_i],
            device_id=xpartner, device_id_type=pl.DeviceIdType.MESH)

        @pl.when(i < R)
        def _():
            x_rdma.start()

        @pl.when(i >= 3)
        def _():
            pl.semaphore_wait(y_credit, 1)

        y_rdma = pltpu.make_async_remote_copy(
            src_ref=xrecv.at[s_p],
            dst_ref=yrecv.at[s_p],
            send_sem=y_send_sem, recv_sem=y_recv_sems.at[s_p],
            device_id=ypartner, device_id_type=pl.DeviceIdType.MESH)

        @pl.when(i > 0)
        def _():
            y_rdma.start()

            yd = (pc_ref[pl.ds(d_off, H), :].astype(jnp.float32)
                  + xrecv[s_p].astype(jnp.float32)
                  + resid_ref[pl.ds(d_off, H), :])
            msd = jnp.mean(yd * yd, axis=-1, keepdims=True)
            out_ref[pl.ds(d_off, H), :] = (
                yd * lax.rsqrt(msd + 1e-6) * gamma_ref[...])

            y_rdma.wait_recv()
            yf = (pc_ref[pl.ds(f_off, H), :].astype(jnp.float32)
                  + yrecv[s_p].astype(jnp.float32)
                  + resid_ref[pl.ds(f_off, H), :])
            msf = jnp.mean(yf * yf, axis=-1, keepdims=True)
            out_ref[pl.ds(f_off, H), :] = (
                yf * lax.rsqrt(msf + 1e-6) * gamma_ref[...])

            y_rdma.wait_send()

        @pl.when(i < R)
        def _():
            x_rdma.wait()

        @pl.when(jnp.logical_and(i >= 1, i <= R - 2))
        def _():
            pl.semaphore_signal(x_credit, inc=1, device_id=xpartner,
                                device_id_type=pl.DeviceIdType.MESH)
            pl.semaphore_signal(y_credit, inc=1, device_id=ypartner,
                                device_id_type=pl.DeviceIdType.MESH)

    return pl.pallas_call(
        body,
        grid=(R + 1,),
        out_shape=jax.ShapeDtypeStruct((M, D), jnp.float32),
        in_specs=[
            pl.BlockSpec((BR, D), lambda i: (jnp.minimum(i, R - 1), 0)),
            pl.BlockSpec((BR, D), lambda i: (jnp.maximum(i - 1, 0), 0)),
            pl.BlockSpec((BR, D), lambda i: (jnp.maximum(i - 1, 0), 0)),
            pl.BlockSpec((1, D), lambda i: (0, 0)),
        ],
        out_specs=pl.BlockSpec((BR, D), lambda i: (jnp.maximum(i - 1, 0), 0)),
        scratch_shapes=[
            pltpu.VMEM((2, H, D), jnp.bfloat16),
            pltpu.VMEM((2, H, D), jnp.bfloat16),
            pltpu.SemaphoreType.DMA,
            pltpu.SemaphoreType.DMA((2,)),
            pltpu.SemaphoreType.DMA,
            pltpu.SemaphoreType.DMA((2,)),
            pltpu.SemaphoreType.REGULAR,
            pltpu.SemaphoreType.REGULAR,
        ],
        compiler_params=pltpu.CompilerParams(collective_id=0),
    )(p, p, resid, gamma2)
